# baseline (speedup 1.0000x reference)
import functools

import jax
import jax.numpy as jnp
from jax import lax
from jax.experimental import pallas as pl
from jax.experimental.pallas import tpu as pltpu
from jax.experimental.pallas import tpu_sc as plsc

_EMBED = 64
_NC, _NS = 2, 16
_NW = _NC * _NS
_CHUNK = 128
_SCALE = 8.0
_L = 16
_NBUF = 4

_BATCH, _SEQ = 4096, 200
_NBB = _BATCH // _CHUNK
_NCHUNK = _SEQ * _NBB
_CPW = _NCHUNK // _NW
_NGRP = _CPW // _NBUF
_SROWS = 8

_mesh = plsc.VectorSubcoreMesh(
    core_axis_name="c", subcore_axis_name="s",
    num_cores=_NC, num_subcores=_NS,
)


@functools.partial(
    pl.kernel,
    out_type=jax.ShapeDtypeStruct((_SEQ, 8, _NBB, 8, _CHUNK), jnp.float32),
    mesh=_mesh,
    scratch_types=[
        pltpu.VMEM((_SROWS, _BATCH), jnp.int32),
        pltpu.VMEM((_NBUF, _CHUNK, _EMBED), jnp.float32),
        pltpu.VMEM((_NBUF, 8, 8, _CHUNK + 1), jnp.float32),
        pltpu.SemaphoreType.DMA((_NBUF,)),
        pltpu.SemaphoreType.DMA((_NBUF,)),
    ],
    compiler_params=pltpu.CompilerParams(
        use_tc_tiling_on_sc=False, needs_layout_passes=False),
)
def _emb_lookup(xt_hbm, table_hbm, out_hbm, idx_v, gbuf, wbuf, gsem, wsem):
    wid = lax.axis_index("s") * _NC + lax.axis_index("c")
    k0 = wid * _CPW
    s_start = jnp.minimum(k0 // _NBB, _SEQ - _SROWS)
    pltpu.sync_copy(xt_hbm.at[pl.ds(s_start, _SROWS)], idx_v)

    def idx_slice(c):
        k = k0 + c
        s, b_hi = k // _NBB, k % _NBB
        return idx_v.at[s - s_start, pl.ds(b_hi * _CHUNK, _CHUNK)]

    def fire_gather(c, b):
        pltpu.async_copy(table_hbm.at[idx_slice(c)], gbuf.at[b], gsem.at[b])

    def wait_gather(c, b):
        pltpu.make_async_copy(table_hbm.at[idx_slice(c)], gbuf.at[b],
                              gsem.at[b]).wait()

    def out_slice(c):
        k = k0 + c
        s, b_hi = k // _NBB, k % _NBB
        return out_hbm.at[s, :, b_hi]

    def fire_write(c, b):
        pltpu.async_copy(wbuf.at[b, :, :, pl.ds(0, _CHUNK)], out_slice(c),
                         wsem.at[b])

    def wait_write(c, b):
        pltpu.make_async_copy(wbuf.at[b, :, :, pl.ds(0, _CHUNK)],
                              out_slice(c), wsem.at[b]).wait()

    iota = lax.iota(jnp.int32, _L)
    e8v = [(iota + l * _L) // 8 for l in range(_EMBED // _L)]
    e_lov = [(iota + l * _L) % 8 for l in range(_EMBED // _L)]

    def transpose_scale(b):
        @plsc.parallel_loop(0, _CHUNK, unroll=2)
        def _row(r):
            rcol = jnp.full((_L,), 0, jnp.int32) + r
            for l in range(_EMBED // _L):
                v = gbuf[b, r, pl.ds(l * _L, _L)]
                plsc.store_scatter(wbuf.at[b], [e8v[l], e_lov[l], rcol],
                                   v * _SCALE)

    for b in range(_NBUF):
        fire_gather(b, b)

    for b in range(_NBUF):
        wait_gather(b, b)
        transpose_scale(b)
        fire_write(b, b)
        fire_gather(_NBUF + b, b)

    def group(g, carry):
        for b in range(_NBUF):
            c = g * _NBUF + b
            wait_gather(c, b)
            wait_write(c - _NBUF, b)
            transpose_scale(b)
            fire_write(c, b)
            fire_gather(c + _NBUF, b)
        return carry

    lax.fori_loop(1, _NGRP - 1, group, 0)

    for b in range(_NBUF):
        c = (_NGRP - 1) * _NBUF + b
        wait_gather(c, b)
        wait_write(c - _NBUF, b)
        transpose_scale(b)
        fire_write(c, b)

    for b in range(_NBUF):
        wait_write((_NGRP - 1) * _NBUF + b, b)


def kernel(x, table):
    o5 = _emb_lookup(x.astype(jnp.int32).T, table)
    return (o5.transpose(2, 4, 0, 1, 3)
            .reshape(_BATCH, _SEQ, _EMBED))


# speedup vs baseline: 1.0015x; 1.0015x over previous
import functools

import jax
import jax.numpy as jnp
from jax import lax
from jax.experimental import pallas as pl
from jax.experimental.pallas import tpu as pltpu
from jax.experimental.pallas import tpu_sc as plsc

_EMBED = 64
_NC, _NS = 2, 16
_NW = _NC * _NS
_CHUNK = 128
_SCALE = 8.0
_L = 16
_NBUF = 4

_BATCH, _SEQ = 4096, 200
_NBB = _BATCH // _CHUNK
_NCHUNK = _SEQ * _NBB
_CPW = _NCHUNK // _NW
_NGRP = _CPW // _NBUF
_SROWS = 8

_mesh = plsc.VectorSubcoreMesh(
    core_axis_name="c", subcore_axis_name="s",
    num_cores=_NC, num_subcores=_NS,
)


@functools.partial(
    pl.kernel,
    out_type=jax.ShapeDtypeStruct((_SEQ, 8, _NBB, 8, _CHUNK), jnp.float32),
    mesh=_mesh,
    scratch_types=[
        pltpu.VMEM((_SROWS, _BATCH), jnp.int32),
        pltpu.VMEM((_NBUF, _CHUNK, _EMBED), jnp.float32),
        pltpu.VMEM((_NBUF, 8, 8, _CHUNK + 1), jnp.float32),
        pltpu.SemaphoreType.DMA((_NBUF,)),
        pltpu.SemaphoreType.DMA((_NBUF,)),
    ],
    compiler_params=pltpu.CompilerParams(
        use_tc_tiling_on_sc=False, needs_layout_passes=False),
)
def _emb_lookup(xt_hbm, table_hbm, out_hbm, idx_v, gbuf, wbuf, gsem, wsem):
    wid = lax.axis_index("s") * _NC + lax.axis_index("c")
    k0 = wid * _CPW
    s_start = jnp.minimum(k0 // _NBB, _SEQ - _SROWS)
    pltpu.sync_copy(xt_hbm.at[pl.ds(s_start, _SROWS)], idx_v)

    def idx_slice(c):
        k = k0 + c
        s, b_hi = k // _NBB, k % _NBB
        return idx_v.at[s - s_start, pl.ds(b_hi * _CHUNK, _CHUNK)]

    def fire_gather(c, b):
        pltpu.async_copy(table_hbm.at[idx_slice(c)], gbuf.at[b], gsem.at[b])

    def wait_gather(c, b):
        pltpu.make_async_copy(table_hbm.at[idx_slice(c)], gbuf.at[b],
                              gsem.at[b]).wait()

    def out_slice(c):
        k = k0 + c
        s, b_hi = k // _NBB, k % _NBB
        return out_hbm.at[s, :, b_hi]

    def fire_write(c, b):
        pltpu.async_copy(wbuf.at[b, :, :, pl.ds(0, _CHUNK)], out_slice(c),
                         wsem.at[b])

    def wait_write(c, b):
        pltpu.make_async_copy(wbuf.at[b, :, :, pl.ds(0, _CHUNK)],
                              out_slice(c), wsem.at[b]).wait()

    iota = lax.iota(jnp.int32, _L)
    e8v = [(iota + l * _L) // 8 for l in range(_EMBED // _L)]
    e_lov = [(iota + l * _L) % 8 for l in range(_EMBED // _L)]

    def transpose_scale(b):
        @plsc.parallel_loop(0, _CHUNK, unroll=2)
        def _row(r):
            rcol = jnp.full((_L,), 0, jnp.int32) + r
            for l in range(_EMBED // _L):
                v = gbuf[b, r, pl.ds(l * _L, _L)]
                plsc.store_scatter(wbuf.at[b], [e8v[l], e_lov[l], rcol],
                                   v * _SCALE)

    for b in range(_NBUF):
        fire_gather(b, b)

    for b in range(_NBUF):
        wait_gather(b, b)
        transpose_scale(b)
        fire_write(b, b)
        fire_gather(_NBUF + b, b)

    def group(g, carry):
        for b in range(_NBUF):
            c = g * _NBUF + b
            wait_gather(c, b)
            wait_write(c - _NBUF, b)
            transpose_scale(b)
            fire_write(c, b)
            fire_gather(c + _NBUF, b)
        return carry

    lax.fori_loop(1, _NGRP - 1, group, 0)

    for b in range(_NBUF):
        c = (_NGRP - 1) * _NBUF + b
        wait_gather(c, b)
        wait_write(c - _NBUF, b)
        transpose_scale(b)
        fire_write(c, b)

    for b in range(_NBUF):
        wait_write((_NGRP - 1) * _NBUF + b, b)


def kernel(x, table):
    xt = x.astype(jnp.float32).T.astype(jnp.int32)
    o5 = _emb_lookup(xt, table)
    return (o5.transpose(2, 4, 0, 1, 3)
            .reshape(_BATCH, _SEQ, _EMBED))


# speedup vs baseline: 1.0031x; 1.0016x over previous
import functools

import jax
import jax.numpy as jnp
from jax import lax
from jax.experimental import pallas as pl
from jax.experimental.pallas import tpu as pltpu
from jax.experimental.pallas import tpu_sc as plsc

_EMBED = 64
_NC, _NS = 2, 16
_NW = _NC * _NS
_CHUNK = 128
_SCALE = 8.0
_L = 16
_NBUF = 4

_BATCH, _SEQ = 4096, 200
_NBB = _BATCH // _CHUNK
_NCHUNK = _SEQ * _NBB
_CPW = _NCHUNK // _NW
_GPW = _CPW // 8

_mesh = plsc.VectorSubcoreMesh(
    core_axis_name="c", subcore_axis_name="s",
    num_cores=_NC, num_subcores=_NS,
)


@functools.partial(
    pl.kernel,
    out_type=jax.ShapeDtypeStruct((_SEQ, 8, _NBB, 8, _CHUNK), jnp.float32),
    mesh=_mesh,
    scratch_types=[
        pltpu.VMEM((2, _CHUNK, 8), jnp.int32),
        pltpu.VMEM((2, 8, _CHUNK), jnp.int32),
        pltpu.VMEM((_NBUF, _CHUNK, _EMBED), jnp.float32),
        pltpu.VMEM((_NBUF, 8, 8, _CHUNK + 1), jnp.float32),
        pltpu.SemaphoreType.DMA((2,)),
        pltpu.SemaphoreType.DMA((_NBUF,)),
        pltpu.SemaphoreType.DMA((_NBUF,)),
    ],
    compiler_params=pltpu.CompilerParams(
        use_tc_tiling_on_sc=False, needs_layout_passes=False),
)
def _emb_lookup(x_hbm, table_hbm, out_hbm, ibuf, tbuf, gbuf, wbuf, isem,
                gsem, wsem):
    wid = lax.axis_index("s") * _NC + lax.axis_index("c")
    g0 = wid * _GPW
    iota = lax.iota(jnp.int32, _L)

    def gsb(g):
        gg = g0 + g
        return gg // _NBB, gg % _NBB

    def xslice(g):
        sblk, b_hi = gsb(g)
        return x_hbm.at[pl.ds(b_hi * _CHUNK, _CHUNK), pl.ds(sblk * 8, 8)]

    def fire_iblock(g):
        pltpu.async_copy(xslice(g), ibuf.at[g % 2], isem.at[g % 2])

    def wait_iblock(g):
        pltpu.make_async_copy(xslice(g), ibuf.at[g % 2],
                              isem.at[g % 2]).wait()

    def transpose_idx(g):
        src, dst = ibuf.at[g % 2], tbuf.at[g % 2]
        for s_lo in range(8):
            scol = jnp.full((_L,), 0, jnp.int32) + s_lo
            for l in range(_CHUNK // _L):
                v = plsc.load_gather(src, [iota + l * _L, scol])
                dst[s_lo, pl.ds(l * _L, _L)] = v

    def fire_gather(c, b):
        g, s_lo = c // 8, c % 8
        pltpu.async_copy(table_hbm.at[tbuf.at[g % 2, s_lo]], gbuf.at[b],
                         gsem.at[b])

    def wait_gather(c, b):
        g, s_lo = c // 8, c % 8
        pltpu.make_async_copy(table_hbm.at[tbuf.at[g % 2, s_lo]],
                              gbuf.at[b], gsem.at[b]).wait()

    def out_slice(c):
        g, s_lo = c // 8, c % 8
        sblk, b_hi = gsb(g)
        return out_hbm.at[sblk * 8 + s_lo, :, b_hi]

    def fire_write(c, b):
        pltpu.async_copy(wbuf.at[b, :, :, pl.ds(0, _CHUNK)], out_slice(c),
                         wsem.at[b])

    def wait_write(c, b):
        pltpu.make_async_copy(wbuf.at[b, :, :, pl.ds(0, _CHUNK)],
                              out_slice(c), wsem.at[b]).wait()

    e8v = [(iota + l * _L) // 8 for l in range(_EMBED // _L)]
    e_lov = [(iota + l * _L) % 8 for l in range(_EMBED // _L)]

    def transpose_scale(b):
        @plsc.parallel_loop(0, _CHUNK, unroll=2)
        def _row(r):
            rcol = jnp.full((_L,), 0, jnp.int32) + r
            for l in range(_EMBED // _L):
                v = gbuf[b, r, pl.ds(l * _L, _L)]
                plsc.store_scatter(wbuf.at[b], [e8v[l], e_lov[l], rcol],
                                   v * _SCALE)

    fire_iblock(0)
    fire_iblock(1)
    wait_iblock(0)
    transpose_idx(0)
    for c in range(3):
        fire_gather(c, c)

    wait_iblock(1)
    transpose_idx(1)
    fire_iblock(2)
    for j in range(8):
        b = j % _NBUF
        wait_gather(j, b)
        fire_gather(j + 3, (j + 3) % _NBUF)
        if j >= _NBUF:
            wait_write(j - _NBUF, b)
        transpose_scale(b)
        fire_write(j, b)

    def group_body(g, carry):
        @pl.when(g + 1 < _GPW)
        def _():
            wait_iblock(g + 1)
            transpose_idx(g + 1)

        @pl.when(g + 2 < _GPW)
        def _():
            fire_iblock(g + 2)

        for j in range(8):
            c = g * 8 + j
            b = j % _NBUF
            wait_gather(c, b)

            @pl.when(c + 3 < _CPW)
            def _():
                fire_gather(c + 3, (j + 3) % _NBUF)

            wait_write(c - _NBUF, b)
            transpose_scale(b)
            fire_write(c, b)
        return carry

    lax.fori_loop(1, _GPW, group_body, 0)

    for b in range(_NBUF):
        wait_write(_CPW - _NBUF + b, b)


def kernel(x, table):
    o5 = _emb_lookup(x.astype(jnp.int32), table)
    return (o5.transpose(2, 4, 0, 1, 3)
            .reshape(_BATCH, _SEQ, _EMBED))
